# baseline (speedup 1.0000x reference)
import functools

import jax
import jax.numpy as jnp
from jax import lax
from jax.experimental import pallas as pl
from jax.experimental.pallas import tpu as pltpu

N_SEG = 1024
WIN = 128
N_WIN = N_SEG // WIN
ROWS = 232
NEG = -1e30


def _pick_block(n):
    for k in (6400, 3200, 2560, 2048, 1536, 1280, 1024, 768, 640, 512, 384, 256, 128):
        if n % k == 0:
            return k
    return n


def _body(nb, k_pts,
          s_ref, v_ref, pos_ref, bid_ref, bounds_ref,
          te_ref,
          a1w_ref, s1wv_ref, s1ws_ref, v1w2_ref, g1wt_ref, g1b_ref, d1w_ref,
          v2w1_ref, s2wv_ref, s2ws_ref, n2v2_ref, g2wt_ref, g2b_ref,
          sd2_ref, bd2_ref, bd2c_ref, bd_ref,
          out_ref, m_ref, den_ref):
    i = pl.program_id(0)

    @pl.when(i == 0)
    def _init():
        out_ref[:] = jnp.zeros_like(out_ref)
        m_ref[:] = jnp.full_like(m_ref, NEG)
        den_ref[:] = jnp.zeros_like(den_ref)

    f32 = jnp.float32
    dot = functools.partial(jnp.dot, preferred_element_type=f32)
    cn0 = (((0,), (0,)), ((), ()))

    def dot0(a, b):
        return lax.dot_general(a, b, cn0, preferred_element_type=f32)

    Sd2 = sd2_ref[:]
    Bd2 = bd2_ref[:]
    Bd = bd_ref[:]

    s0 = s_ref[:] + te_ref[:]
    v0 = v_ref[:]

    viT = dot(a1w_ref[:], v0)
    vn = jnp.sqrt(dot(Sd2, viT * viT))
    os_ = dot0(vn, s1wv_ref[:]) + dot(s0, s1ws_ref[:])
    ovT = dot(v1w2_ref[:], viT)
    gate = jax.nn.sigmoid(dot(os_, g1wt_ref[:]) + g1b_ref[:])
    v1 = dot(Bd2, gate.T) * ovT

    dT = dot(d1w_ref[:], v1)
    dt = dot(Sd2, v1 * dT)
    dsq = dot(Sd2, dT * dT)
    coef = jnp.where(dt >= 0.0, 0.0, dt / (dsq + 1e-9))
    v2 = v1 - dot(bd2c_ref[:], coef) * dT
    s_act = jnp.maximum(os_, 0.01 * os_)

    vi2T = dot(v2w1_ref[:], v2)
    vn2 = jnp.sqrt(dot(Sd2, vi2T * vi2T))
    salin = dot0(vn2, s2wv_ref[:]) + dot(s_act, s2ws_ref[:])
    att_b = salin[:, 0:128]
    hs = salin[:, 128:256]
    ov2T = dot(n2v2_ref[:], vi2T[96:192, :])
    gate2 = jax.nn.sigmoid(dot(hs, g2wt_ref[:]) + g2b_ref[:])
    hvT = dot(Bd, gate2.T) * ov2T
    vpT = jnp.concatenate(
        [hvT, pos_ref[:], jnp.zeros((5, k_pts), dtype=f32)], axis=0)

    w2 = jnp.concatenate([jnp.full((1, WIN), 32.0, dtype=f32),
                          jnp.ones((1, WIN), dtype=f32)], axis=0)
    bid_b = dot0(bid_ref[:], w2)
    bid_lo = bounds_ref[0, 0, 0]
    bid_hi = bounds_ref[0, 0, 1]
    lane = lax.broadcasted_iota(jnp.int32, (1, WIN), 1).astype(f32)

    for w in range(N_WIN):
        base = w * WIN

        @pl.when((bid_hi >= base) & (bid_lo < base + WIN))
        def _win(w=w, base=base):
            O = bid_b == (float(base) + lane)
            att_m = jnp.where(O, att_b, NEG)
            m_old = m_ref[w:w + 1, :]
            m_new = jnp.maximum(m_old, jnp.max(att_m, axis=0, keepdims=True))
            scale = jnp.exp(m_old - m_new)
            p = jnp.where(O, jnp.exp(att_m - m_new), 0.0)
            m_ref[w:w + 1, :] = m_new
            den_ref[w:w + 1, :] = (den_ref[w:w + 1, :] * scale
                                   + jnp.sum(p, axis=0, keepdims=True))
            part_hs = dot0(hs, p)
            part_vp = dot(vpT, p)
            sl = slice(base, base + WIN)
            out_ref[0:128, sl] = out_ref[0:128, sl] * scale + part_hs
            out_ref[128:232, sl] = out_ref[128:232, sl] * scale + part_vp

    @pl.when(i == nb - 1)
    def _fin():
        den = den_ref[:]
        den_safe = jnp.where(den == 0.0, 1.0, den)
        for w in range(N_WIN):
            sl = slice(w * WIN, (w + 1) * WIN)
            out_ref[:, sl] = out_ref[:, sl] / den_safe[w:w + 1, :]


def kernel(h_att_sca, h_att_vec, pos_context, batch_id, t, params):
    n = h_att_sca.shape[0]
    hv_ch = h_att_vec.shape[1]
    k_pts = _pick_block(n)
    nb = n // k_pts
    f32 = jnp.float32

    eye3 = jnp.eye(3, dtype=f32)
    eyec = jnp.eye(hv_ch, dtype=f32)

    def kron3(W):
        return jnp.kron(eye3, W)

    def bdiag(a, b):
        z1 = jnp.zeros((a.shape[0], b.shape[1]), dtype=f32)
        z2 = jnp.zeros((b.shape[0], a.shape[1]), dtype=f32)
        return jnp.block([[a, z1], [z2, b]])

    p = params
    sd = jnp.kron(jnp.ones((1, 3), dtype=f32), eyec)
    bd = jnp.kron(jnp.ones((3, 1), dtype=f32), eyec)

    te = p['time_embed'][t][None, :]
    v_t = h_att_vec.transpose(2, 1, 0).reshape(3 * hv_ch, n)
    pos_t = pos_context.T
    bidi = batch_id.astype(jnp.int32)
    bid2 = jnp.stack([(bidi >> 5).astype(f32),
                      (bidi & 31).astype(f32)], axis=0)
    bounds = jnp.stack([bidi[::k_pts], bidi[k_pts - 1::k_pts]],
                       axis=1).reshape(nb, 1, 2)

    args = [
        h_att_sca, v_t, pos_t, bid2, bounds,
        te,
        jnp.concatenate([kron3(p['a1_vW1']), kron3(p['n1_vW1'])], axis=0),
        bdiag(p['a1_sW'][:, :hv_ch].T, p['n1_sW'][:, :hv_ch].T),
        jnp.concatenate([p['a1_sW'][:, hv_ch:].T,
                         p['n1_sW'][:, hv_ch:].T], axis=1),
        bdiag(kron3(p['a1_vW2']), kron3(p['n1_vW2'])),
        bdiag(p['a1_gW'].T, p['n1_gW'].T),
        jnp.concatenate([p['a1_gb'], p['n1_gb']])[None, :],
        bdiag(kron3(p['a1_dW']), kron3(p['n1_dW'])),
        bdiag(kron3(p['a2_vW1']), kron3(p['n2_vW1'])),
        bdiag(jnp.tile(p['a2_sW'][:, :hv_ch].T, (1, WIN)),
              p['n2_sW'][:, :hv_ch].T),
        bdiag(jnp.tile(p['a2_sW'][:, hv_ch:].T, (1, WIN)),
              p['n2_sW'][:, hv_ch:].T),
        kron3(p['n2_vW2']),
        p['n2_gW'].T, p['n2_gb'][None, :],
        bdiag(sd, sd), bdiag(bd, bd), 0.8 * bdiag(bd, bd), bd,
    ]

    def fixed(a):
        shape = a.shape
        return pl.BlockSpec(shape, lambda i: (0,) * len(shape))

    in_specs = [
        pl.BlockSpec((k_pts, 128), lambda i: (i, 0)),
        pl.BlockSpec((3 * hv_ch, k_pts), lambda i: (0, i)),
        pl.BlockSpec((3, k_pts), lambda i: (0, i)),
        pl.BlockSpec((2, k_pts), lambda i: (0, i)),
        pl.BlockSpec((1, 1, 2), lambda i: (i, 0, 0), memory_space=pltpu.SMEM),
    ] + [fixed(a) for a in args[5:]]

    out = pl.pallas_call(
        functools.partial(_body, nb, k_pts),
        grid=(nb,),
        in_specs=in_specs,
        out_specs=pl.BlockSpec((ROWS, N_SEG), lambda i: (0, 0)),
        out_shape=jax.ShapeDtypeStruct((ROWS, N_SEG), f32),
        scratch_shapes=[
            pltpu.VMEM((N_WIN, WIN), f32),
            pltpu.VMEM((N_WIN, WIN), f32),
        ],
        compiler_params=pltpu.CompilerParams(
            dimension_semantics=("arbitrary",)),
    )(*args)

    feat = out[0:128, :].T
    vec = out[128:128 + 3 * hv_ch, :].reshape(3, hv_ch, N_SEG).transpose(2, 1, 0)
    pos = out[224:227, :].T
    return feat, vec, pos


# speedup vs baseline: 1.0341x; 1.0341x over previous
import functools

import jax
import jax.numpy as jnp
from jax import lax
from jax.experimental import pallas as pl
from jax.experimental.pallas import tpu as pltpu

N_SEG = 1024
WIN = 128
N_WIN = N_SEG // WIN
ROWS = 232
NEG = -1e30


def _pick_block(n):
    for k in (3200, 2560, 2048, 1536, 1280, 1024, 768, 640, 512, 384, 256, 128):
        if n % k == 0:
            return k
    return n


def _body(nb, k_pts,
          s_ref, v_ref, pos_ref, bid_ref, bounds_ref,
          te_ref,
          a1w_ref, s1wv_ref, s1ws_ref, v1w2_ref, g1wt_ref, g1b_ref, d1w_ref,
          v2w1_ref, s2wv_ref, s2ws_ref, n2v2_ref, g2wt_ref, g2b_ref,
          sd2_ref, bd2_ref, bd2c_ref, bd_ref,
          out_ref, m_ref, den_ref):
    i = pl.program_id(0)

    @pl.when(i == 0)
    def _init():
        out_ref[:] = jnp.zeros_like(out_ref)
        m_ref[:] = jnp.full_like(m_ref, NEG)
        den_ref[:] = jnp.zeros_like(den_ref)

    f32 = jnp.float32
    dot = functools.partial(jnp.dot, preferred_element_type=f32)
    cn0 = (((0,), (0,)), ((), ()))

    def dot0(a, b):
        return lax.dot_general(a, b, cn0, preferred_element_type=f32)

    Sd2 = sd2_ref[:]
    Bd2 = bd2_ref[:]
    Bd = bd_ref[:]

    s0 = s_ref[:] + te_ref[:]
    v0 = v_ref[:]

    viT = dot(a1w_ref[:], v0)
    vn = jnp.sqrt(dot(Sd2, viT * viT))
    os_ = dot0(vn, s1wv_ref[:]) + dot(s0, s1ws_ref[:])
    ovT = dot(v1w2_ref[:], viT)
    gate = jax.nn.sigmoid(dot(os_, g1wt_ref[:]) + g1b_ref[:])
    v1 = dot(Bd2, gate.T) * ovT

    dT = dot(d1w_ref[:], v1)
    dt = dot(Sd2, v1 * dT)
    dsq = dot(Sd2, dT * dT)
    coef = jnp.where(dt >= 0.0, 0.0, dt / (dsq + 1e-9))
    v2 = v1 - dot(bd2c_ref[:], coef) * dT
    s_act = jnp.maximum(os_, 0.01 * os_)

    vi2T = dot(v2w1_ref[:], v2)
    vn2 = jnp.sqrt(dot(Sd2, vi2T * vi2T))
    salin = dot0(vn2, s2wv_ref[:]) + dot(s_act, s2ws_ref[:])
    att_b = salin[:, 0:128]
    hs = salin[:, 128:256]
    ov2T = dot(n2v2_ref[:], vi2T[96:192, :])
    gate2 = jax.nn.sigmoid(dot(hs, g2wt_ref[:]) + g2b_ref[:])
    hvT = dot(Bd, gate2.T) * ov2T
    vpT = jnp.concatenate(
        [hvT, pos_ref[:], jnp.zeros((5, k_pts), dtype=f32)], axis=0)

    w2 = jnp.concatenate([jnp.full((1, WIN), 32.0, dtype=f32),
                          jnp.ones((1, WIN), dtype=f32)], axis=0)
    bid_b = dot0(bid_ref[:], w2)
    bid_lo = bounds_ref[0, 0, 0]
    bid_hi = bounds_ref[0, 0, 1]
    lane = lax.broadcasted_iota(jnp.int32, (1, WIN), 1).astype(f32)

    for w in range(N_WIN):
        base = w * WIN

        @pl.when((bid_hi >= base) & (bid_lo < base + WIN))
        def _win(w=w, base=base):
            O = bid_b == (float(base) + lane)
            att_m = jnp.where(O, att_b, NEG)
            m_old = m_ref[w:w + 1, :]
            m_new = jnp.maximum(m_old, jnp.max(att_m, axis=0, keepdims=True))
            scale = jnp.exp(m_old - m_new)
            p = jnp.exp(att_m - m_new)
            m_ref[w:w + 1, :] = m_new
            den_ref[w:w + 1, :] = (den_ref[w:w + 1, :] * scale
                                   + jnp.sum(p, axis=0, keepdims=True))
            part_hs = dot0(hs, p)
            part_vp = dot(vpT, p)
            sl = slice(base, base + WIN)
            out_ref[0:128, sl] = out_ref[0:128, sl] * scale + part_hs
            out_ref[128:232, sl] = out_ref[128:232, sl] * scale + part_vp

    @pl.when(i == nb - 1)
    def _fin():
        den = den_ref[:]
        den_safe = jnp.where(den == 0.0, 1.0, den)
        m_fin = m_ref[:]
        for w in range(N_WIN):
            sl = slice(w * WIN, (w + 1) * WIN)
            seen = m_fin[w:w + 1, :] > NEG
            out_ref[:, sl] = jnp.where(
                seen, out_ref[:, sl] / den_safe[w:w + 1, :], 0.0)


def kernel(h_att_sca, h_att_vec, pos_context, batch_id, t, params):
    n = h_att_sca.shape[0]
    hv_ch = h_att_vec.shape[1]
    k_pts = _pick_block(n)
    nb = n // k_pts
    f32 = jnp.float32

    eye3 = jnp.eye(3, dtype=f32)
    eyec = jnp.eye(hv_ch, dtype=f32)

    def kron3(W):
        return jnp.kron(eye3, W)

    def bdiag(a, b):
        z1 = jnp.zeros((a.shape[0], b.shape[1]), dtype=f32)
        z2 = jnp.zeros((b.shape[0], a.shape[1]), dtype=f32)
        return jnp.block([[a, z1], [z2, b]])

    p = params
    sd = jnp.kron(jnp.ones((1, 3), dtype=f32), eyec)
    bd = jnp.kron(jnp.ones((3, 1), dtype=f32), eyec)

    te = p['time_embed'][t][None, :]
    v_t = h_att_vec.transpose(2, 1, 0).reshape(3 * hv_ch, n)
    pos_t = pos_context.T
    bidi = batch_id.astype(jnp.int32)
    bid2 = jnp.stack([(bidi >> 5).astype(f32),
                      (bidi & 31).astype(f32)], axis=0)
    bounds = jnp.stack([bidi[::k_pts], bidi[k_pts - 1::k_pts]],
                       axis=1).reshape(nb, 1, 2)

    args = [
        h_att_sca, v_t, pos_t, bid2, bounds,
        te,
        jnp.concatenate([kron3(p['a1_vW1']), kron3(p['n1_vW1'])], axis=0),
        bdiag(p['a1_sW'][:, :hv_ch].T, p['n1_sW'][:, :hv_ch].T),
        jnp.concatenate([p['a1_sW'][:, hv_ch:].T,
                         p['n1_sW'][:, hv_ch:].T], axis=1),
        bdiag(kron3(p['a1_vW2']), kron3(p['n1_vW2'])),
        bdiag(p['a1_gW'].T, p['n1_gW'].T),
        jnp.concatenate([p['a1_gb'], p['n1_gb']])[None, :],
        bdiag(kron3(p['a1_dW']), kron3(p['n1_dW'])),
        bdiag(kron3(p['a2_vW1']), kron3(p['n2_vW1'])),
        bdiag(jnp.tile(p['a2_sW'][:, :hv_ch].T, (1, WIN)),
              p['n2_sW'][:, :hv_ch].T),
        bdiag(jnp.tile(p['a2_sW'][:, hv_ch:].T, (1, WIN)),
              p['n2_sW'][:, hv_ch:].T),
        kron3(p['n2_vW2']),
        p['n2_gW'].T, p['n2_gb'][None, :],
        bdiag(sd, sd), bdiag(bd, bd), 0.8 * bdiag(bd, bd), bd,
    ]

    def fixed(a):
        shape = a.shape
        return pl.BlockSpec(shape, lambda i: (0,) * len(shape))

    in_specs = [
        pl.BlockSpec((k_pts, 128), lambda i: (i, 0)),
        pl.BlockSpec((3 * hv_ch, k_pts), lambda i: (0, i)),
        pl.BlockSpec((3, k_pts), lambda i: (0, i)),
        pl.BlockSpec((2, k_pts), lambda i: (0, i)),
        pl.BlockSpec((1, 1, 2), lambda i: (i, 0, 0), memory_space=pltpu.SMEM),
    ] + [fixed(a) for a in args[5:]]

    out = pl.pallas_call(
        functools.partial(_body, nb, k_pts),
        grid=(nb,),
        in_specs=in_specs,
        out_specs=pl.BlockSpec((ROWS, N_SEG), lambda i: (0, 0)),
        out_shape=jax.ShapeDtypeStruct((ROWS, N_SEG), f32),
        scratch_shapes=[
            pltpu.VMEM((N_WIN, WIN), f32),
            pltpu.VMEM((N_WIN, WIN), f32),
        ],
        compiler_params=pltpu.CompilerParams(
            dimension_semantics=("arbitrary",)),
    )(*args)

    feat = out[0:128, :].T
    vec = out[128:128 + 3 * hv_ch, :].reshape(3, hv_ch, N_SEG).transpose(2, 1, 0)
    pos = out[224:227, :].T
    return feat, vec, pos
